# baseline (speedup 1.0000x reference)
import functools

import jax
import jax.numpy as jnp
from jax import lax
from jax.experimental import pallas as pl
from jax.experimental.pallas import tpu as pltpu
from jax.experimental.pallas import tpu_sc as plsc

NC = 2
NS = 16
L = 16
CH = 128
KB = 4
NQ = 4


def _matmul_body(x_ref, w_ref, o_ref):
    o_ref[...] = jnp.dot(x_ref[...], w_ref[...], preferred_element_type=jnp.float32)


def _matmul(xT, W, bn):
    tn = xT.shape[0]
    return pl.pallas_call(
        _matmul_body,
        grid=(tn // bn,),
        in_specs=[
            pl.BlockSpec((bn, xT.shape[1]), lambda i: (i, 0)),
            pl.BlockSpec(W.shape, lambda i: (0, 0)),
        ],
        out_specs=pl.BlockSpec((bn, W.shape[1]), lambda i: (i, 0)),
        out_shape=jax.ShapeDtypeStruct((tn, W.shape[1]), jnp.float32),
    )(xT, W)


def _rsqrt16(d):
    i = lax.bitcast_convert_type(d, jnp.int32)
    i = jnp.int32(0x5F3759DF) - (i >> 1)
    y = lax.bitcast_convert_type(i, jnp.float32)
    for _ in range(3):
        y = y * (1.5 - 0.5 * d * y * y)
    return y


def _bcast(v, i):
    return lax.gather(
        v, jnp.full((L, 1), i, jnp.int32),
        lax.GatherDimensionNumbers(offset_dims=(), collapsed_slice_dims=(0,),
                                   start_index_map=(0,)),
        (1,), mode=lax.GatherScatterMode.PROMISE_IN_BOUNDS)


def _sc_body(n, npad, nacc, nchunk, t_per_core,
             h_ref, src_ref, dst_ref, w_ref, b_ref, out_ref,
             ev_src, ev_dst, ev_wn, deg_v, dis_v, tmp_v, sum_v, b_v, btile,
             rbufs, ev_gi, gsems, ssem, deg_all, dis_sh, acc):
    c = lax.axis_index("c")
    s = lax.axis_index("s")
    rpt = npad // NS
    rbase = s * rpt
    rpta = nacc // NS
    rbasea = s * rpta
    hdim = btile.shape[1]
    nch = hdim // L

    def _zero_deg(i, _):
        deg_v[pl.ds(i * L, L)] = jnp.zeros((L,), jnp.float32)
        return 0
    lax.fori_loop(0, npad // L, _zero_deg, 0)

    pltpu.sync_copy(src_ref.at[s], ev_src)
    pltpu.sync_copy(dst_ref.at[s], ev_dst)
    pltpu.sync_copy(w_ref.at[s], ev_wn)

    def _deg(j, _):
        for k in range(CH // L):
            dv = ev_dst[j, pl.ds(k * L, L)]
            wv = ev_wn[j, pl.ds(k * L, L)]
            plsc.addupdate_scatter(deg_v, [dv], wv)
        return 0
    lax.fori_loop(0, nchunk, _deg, 0)

    pltpu.sync_copy(deg_v, deg_all.at[c, s])
    plsc.subcore_barrier()

    def _zero_sum(i, _):
        sum_v[pl.ds(i * L, L)] = jnp.zeros((L,), jnp.float32)
        return 0
    lax.fori_loop(0, rpt // L, _zero_sum, 0)

    for k in range(NS):
        pltpu.sync_copy(deg_all.at[c, k, pl.ds(rbase, rpt)], tmp_v)

        def _acc_sum(i, _):
            sum_v[pl.ds(i * L, L)] = sum_v[pl.ds(i * L, L)] + tmp_v[pl.ds(i * L, L)]
            return 0
        lax.fori_loop(0, rpt // L, _acc_sum, 0)

    def _dis(i, _):
        tmp_v[pl.ds(i * L, L)] = _rsqrt16(sum_v[pl.ds(i * L, L)])
        return 0
    lax.fori_loop(0, rpt // L, _dis, 0)
    pltpu.sync_copy(tmp_v, dis_sh.at[c, pl.ds(rbase, rpt)])
    plsc.subcore_barrier()
    pltpu.sync_copy(dis_sh.at[c], dis_v)

    def _norm(j, _):
        for k in range(CH // L):
            sv = ev_src[j, pl.ds(k * L, L)]
            dv = ev_dst[j, pl.ds(k * L, L)]
            wv = ev_wn[j, pl.ds(k * L, L)]
            a = plsc.load_gather(dis_v, [sv])
            bb = plsc.load_gather(dis_v, [dv])
            ev_wn[j, pl.ds(k * L, L)] = a * wv * bb
        return 0
    lax.fori_loop(0, nchunk, _norm, 0)

    pltpu.sync_copy(b_ref, b_v)
    nbody = nchunk // KB

    def _pass(tt, _carry):
        t = c * t_per_core + tt
        for hh in range(NQ):
            def _btile(r, _):
                for cc in range(nch):
                    btile[r, pl.ds(cc * L, L)] = b_v[pl.ds(hh * hdim + cc * L, L)]
                return 0
            lax.fori_loop(0, btile.shape[0], _btile, 0)

            brows = btile.shape[0]
            nfull, rem = divmod(rpta, brows)
            for blk in range(nfull):
                pltpu.sync_copy(
                    btile, acc.at[pl.ds(rbasea + blk * brows, brows), :])
            if rem:
                pltpu.sync_copy(
                    btile.at[pl.ds(0, rem), :],
                    acc.at[pl.ds(rbasea + nfull * brows, rem), :])
            plsc.subcore_barrier()

            goff = (t * n) * NQ + hh

            def _scale(kk, j):
                def _rows(g, _):
                    nv = ev_wn[j, pl.ds(g * L, L)]
                    for i in range(L):
                        r = g * L + i
                        bc = _bcast(nv, i)
                        for cc in range(nch):
                            rbufs[kk, r, pl.ds(cc * L, L)] = (
                                rbufs[kk, r, pl.ds(cc * L, L)] * bc)
                    return 0
                lax.fori_loop(0, CH // L, _rows, 0)

            def _mkgidx(kk, j):
                def _g(g, _):
                    sv = ev_src[j, pl.ds(g * L, L)]
                    ev_gi[kk, pl.ds(g * L, L)] = sv * NQ + goff
                    return 0
                lax.fori_loop(0, CH // L, _g, 0)

            def _body(ib, _):
                j0 = ib * KB
                gds = []
                for kk in range(KB):
                    _mkgidx(kk, j0 + kk)
                    gds.append(pltpu.async_copy(
                        h_ref.at[ev_gi.at[kk]], rbufs.at[kk], gsems[kk]))
                sds = []
                for kk in range(KB):
                    gds[kk].wait()
                    _scale(kk, j0 + kk)
                    sds.append(pltpu.async_copy(
                        rbufs.at[kk], acc.at[ev_dst.at[j0 + kk]], ssem,
                        add=True))
                for d in sds:
                    d.wait()
                return 0

            lax.fori_loop(0, nbody, _body, 0)
            plsc.subcore_barrier()

            pltpu.sync_copy(acc.at[pl.ds(rbasea, rpta), :],
                            out_ref.at[t, hh, pl.ds(rbasea, rpta), :])
        return 0
    lax.fori_loop(0, t_per_core, _pass, 0)


def _sc_conv(H2, src3, dst3, w3, b, n, npad, nacc, nchunk, t_all):
    t_per_core = t_all // NC
    cdim = b.shape[0]
    hdim = cdim // NQ
    mesh = plsc.VectorSubcoreMesh(core_axis_name="c", subcore_axis_name="s",
                                  num_cores=NC, num_subcores=NS)
    kern = pl.kernel(
        functools.partial(_sc_body, n, npad, nacc, nchunk, t_per_core),
        out_type=jax.ShapeDtypeStruct((t_all, NQ, nacc, hdim), jnp.float32),
        mesh=mesh,
        compiler_params=pltpu.CompilerParams(
            needs_layout_passes=False, use_tc_tiling_on_sc=False),
        scratch_types=[
            pltpu.VMEM((nchunk, CH), jnp.int32),
            pltpu.VMEM((nchunk, CH), jnp.int32),
            pltpu.VMEM((nchunk, CH), jnp.float32),
            pltpu.VMEM((npad,), jnp.float32),
            pltpu.VMEM((npad,), jnp.float32),
            pltpu.VMEM((npad // NS,), jnp.float32),
            pltpu.VMEM((npad // NS,), jnp.float32),
            pltpu.VMEM((cdim,), jnp.float32),
            pltpu.VMEM((64, cdim // NQ), jnp.float32),
            pltpu.VMEM((KB, CH, cdim // NQ), jnp.float32),
            pltpu.VMEM((KB, CH), jnp.int32),
            [pltpu.SemaphoreType.DMA] * KB,
            pltpu.SemaphoreType.DMA,
            pltpu.HBM((NC, NS, npad), jnp.float32),
            pltpu.HBM((NC, npad), jnp.float32),
            pltpu.VMEM_SHARED((nacc, cdim // NQ), jnp.float32),
        ],
    )
    return kern(H2, src3, dst3, w3, b)


def kernel(x, edge_index, edge_attr, W, b):
    n, c_in, t_all = x.shape
    e = edge_index.shape[1]

    xT = jnp.transpose(x, (2, 0, 1)).reshape(t_all * n, c_in)
    H = _matmul(xT, W, 2000)

    loop = jnp.arange(n, dtype=edge_index.dtype)
    e_tot = e + n
    grp = NS * CH * KB
    e_pad = -(-e_tot // grp) * grp
    pad = e_pad - e_tot
    src = jnp.concatenate([edge_index[0], loop, jnp.zeros((pad,), jnp.int32)])
    dst = jnp.concatenate([edge_index[1], loop, jnp.zeros((pad,), jnp.int32)])
    w = jnp.concatenate([edge_attr, jnp.ones((n,), jnp.float32),
                         jnp.zeros((pad,), jnp.float32)])
    nchunk = e_pad // (NS * CH)
    src3 = src.reshape(NS, nchunk, CH)
    dst3 = dst.reshape(NS, nchunk, CH)
    w3 = w.reshape(NS, nchunk, CH)

    npad = -(-n // 1024) * 1024
    nacc = -(-(n + 1) // 16) * 16
    c_out = W.shape[1]
    H2 = H.reshape(t_all * n * NQ, c_out // NQ)
    out_sc = _sc_conv(H2, src3, dst3, w3, b, n, npad, nacc, nchunk, t_all)
    out = jnp.transpose(out_sc[:, :, :n, :], (2, 1, 3, 0))
    return out.reshape(n, c_out, t_all)


# speedup vs baseline: 1.2579x; 1.2579x over previous
import functools

import jax
import jax.numpy as jnp
from jax import lax
from jax.experimental import pallas as pl
from jax.experimental.pallas import tpu as pltpu
from jax.experimental.pallas import tpu_sc as plsc

NC = 2
NS = 16
L = 16
CH = 128
KB = 3
NQ = 4


def _matmul_body(x_ref, w_ref, o_ref):
    o_ref[...] = jnp.dot(x_ref[...], w_ref[...], preferred_element_type=jnp.float32)


def _matmul(xT, W, bn):
    tn = xT.shape[0]
    return pl.pallas_call(
        _matmul_body,
        grid=(tn // bn,),
        in_specs=[
            pl.BlockSpec((bn, xT.shape[1]), lambda i: (i, 0)),
            pl.BlockSpec(W.shape, lambda i: (0, 0)),
        ],
        out_specs=pl.BlockSpec((bn, W.shape[1]), lambda i: (i, 0)),
        out_shape=jax.ShapeDtypeStruct((tn, W.shape[1]), jnp.float32),
    )(xT, W)


def _rsqrt16(d):
    i = lax.bitcast_convert_type(d, jnp.int32)
    i = jnp.int32(0x5F3759DF) - (i >> 1)
    y = lax.bitcast_convert_type(i, jnp.float32)
    for _ in range(3):
        y = y * (1.5 - 0.5 * d * y * y)
    return y


def _bcast(v, i):
    return lax.gather(
        v, jnp.full((L, 1), i, jnp.int32),
        lax.GatherDimensionNumbers(offset_dims=(), collapsed_slice_dims=(0,),
                                   start_index_map=(0,)),
        (1,), mode=lax.GatherScatterMode.PROMISE_IN_BOUNDS)


def _sc_body(n, npad, nacc, nchunk, t_per_core,
             h_ref, src_ref, dst_ref, w_ref, b_ref, out_ref,
             ev_src, ev_dst, ev_wn, deg_v, dis_v, tmp_v, sum_v, b_v, btile,
             rbufs, ev_gi, gsems, ssem, deg_all, dis_sh, acc):
    c = lax.axis_index("c")
    s = lax.axis_index("s")
    rpt = npad // NS
    rbase = s * rpt
    rpta = nacc // NS
    rbasea = s * rpta
    hdim = btile.shape[1]
    nch = hdim // L

    def _zero_deg(i, _):
        deg_v[pl.ds(i * L, L)] = jnp.zeros((L,), jnp.float32)
        return 0
    lax.fori_loop(0, npad // L, _zero_deg, 0)

    pltpu.sync_copy(src_ref.at[s], ev_src)
    pltpu.sync_copy(dst_ref.at[s], ev_dst)
    pltpu.sync_copy(w_ref.at[s], ev_wn)

    def _deg(j, _):
        for k in range(CH // L):
            dv = ev_dst[j, pl.ds(k * L, L)]
            wv = ev_wn[j, pl.ds(k * L, L)]
            plsc.addupdate_scatter(deg_v, [dv], wv)
        return 0
    lax.fori_loop(0, nchunk, _deg, 0)

    pltpu.sync_copy(deg_v, deg_all.at[c, s])
    plsc.subcore_barrier()

    def _zero_sum(i, _):
        sum_v[pl.ds(i * L, L)] = jnp.zeros((L,), jnp.float32)
        return 0
    lax.fori_loop(0, rpt // L, _zero_sum, 0)

    for k in range(NS):
        pltpu.sync_copy(deg_all.at[c, k, pl.ds(rbase, rpt)], tmp_v)

        def _acc_sum(i, _):
            sum_v[pl.ds(i * L, L)] = sum_v[pl.ds(i * L, L)] + tmp_v[pl.ds(i * L, L)]
            return 0
        lax.fori_loop(0, rpt // L, _acc_sum, 0)

    def _dis(i, _):
        tmp_v[pl.ds(i * L, L)] = _rsqrt16(sum_v[pl.ds(i * L, L)])
        return 0
    lax.fori_loop(0, rpt // L, _dis, 0)
    pltpu.sync_copy(tmp_v, dis_sh.at[c, pl.ds(rbase, rpt)])
    plsc.subcore_barrier()
    pltpu.sync_copy(dis_sh.at[c], dis_v)

    def _norm(j, _):
        for k in range(CH // L):
            sv = ev_src[j, pl.ds(k * L, L)]
            dv = ev_dst[j, pl.ds(k * L, L)]
            wv = ev_wn[j, pl.ds(k * L, L)]
            a = plsc.load_gather(dis_v, [sv])
            bb = plsc.load_gather(dis_v, [dv])
            ev_wn[j, pl.ds(k * L, L)] = a * wv * bb
        return 0
    lax.fori_loop(0, nchunk, _norm, 0)

    pltpu.sync_copy(b_ref, b_v)
    nbody = nchunk // KB

    def _pass(tt, _carry):
        t = c * t_per_core + tt
        for hh in range(NQ):
            def _btile(r, _):
                for cc in range(nch):
                    btile[r, pl.ds(cc * L, L)] = b_v[pl.ds(hh * hdim + cc * L, L)]
                return 0
            lax.fori_loop(0, btile.shape[0], _btile, 0)

            brows = btile.shape[0]
            nfull, rem = divmod(rpta, brows)
            for blk in range(nfull):
                pltpu.sync_copy(
                    btile, acc.at[pl.ds(rbasea + blk * brows, brows), :])
            if rem:
                pltpu.sync_copy(
                    btile.at[pl.ds(0, rem), :],
                    acc.at[pl.ds(rbasea + nfull * brows, rem), :])
            plsc.subcore_barrier()

            goff = (t * n) * NQ + hh

            def _scale(kk, j):
                def _rows(g, _):
                    nv = ev_wn[j, pl.ds(g * L, L)]
                    for i in range(L):
                        r = g * L + i
                        bc = _bcast(nv, i)
                        for cc in range(nch):
                            rbufs[kk, r, pl.ds(cc * L, L)] = (
                                rbufs[kk, r, pl.ds(cc * L, L)] * bc)
                    return 0
                lax.fori_loop(0, CH // L, _rows, 0)

            def _mkgidx(kk, j):
                def _g(g, _):
                    sv = ev_src[j, pl.ds(g * L, L)]
                    ev_gi[kk, pl.ds(g * L, L)] = sv * NQ + goff
                    return 0
                lax.fori_loop(0, CH // L, _g, 0)

            def _body(ib, _):
                j0 = ib * KB
                gds = []
                for kk in range(KB):
                    _mkgidx(kk, j0 + kk)
                    gds.append(pltpu.async_copy(
                        h_ref.at[ev_gi.at[kk]], rbufs.at[kk], gsems[kk]))
                sds = []
                for kk in range(KB):
                    gds[kk].wait()
                    _scale(kk, j0 + kk)
                    sds.append(pltpu.async_copy(
                        rbufs.at[kk], acc.at[ev_dst.at[j0 + kk]], ssem,
                        add=True))
                for d in sds:
                    d.wait()
                return 0

            lax.fori_loop(0, nbody, _body, 0)
            plsc.subcore_barrier()

            pltpu.sync_copy(acc.at[pl.ds(rbasea, rpta), :],
                            out_ref.at[t, hh, pl.ds(rbasea, rpta), :])
        return 0
    lax.fori_loop(0, t_per_core, _pass, 0)


def _sc_conv(H2, src3, dst3, w3, b, n, npad, nacc, nchunk, t_all):
    t_per_core = t_all // NC
    cdim = b.shape[0]
    hdim = cdim // NQ
    mesh = plsc.VectorSubcoreMesh(core_axis_name="c", subcore_axis_name="s",
                                  num_cores=NC, num_subcores=NS)
    kern = pl.kernel(
        functools.partial(_sc_body, n, npad, nacc, nchunk, t_per_core),
        out_type=jax.ShapeDtypeStruct((t_all, NQ, nacc, hdim), jnp.float32),
        mesh=mesh,
        compiler_params=pltpu.CompilerParams(
            needs_layout_passes=False, use_tc_tiling_on_sc=False),
        scratch_types=[
            pltpu.VMEM((nchunk, CH), jnp.int32),
            pltpu.VMEM((nchunk, CH), jnp.int32),
            pltpu.VMEM((nchunk, CH), jnp.float32),
            pltpu.VMEM((npad,), jnp.float32),
            pltpu.VMEM((npad,), jnp.float32),
            pltpu.VMEM((npad // NS,), jnp.float32),
            pltpu.VMEM((npad // NS,), jnp.float32),
            pltpu.VMEM((cdim,), jnp.float32),
            pltpu.VMEM((64, cdim // NQ), jnp.float32),
            pltpu.VMEM((KB, CH, cdim // NQ), jnp.float32),
            pltpu.VMEM((KB, CH), jnp.int32),
            [pltpu.SemaphoreType.DMA] * KB,
            pltpu.SemaphoreType.DMA,
            pltpu.HBM((NC, NS, npad), jnp.float32),
            pltpu.HBM((NC, npad), jnp.float32),
            pltpu.VMEM_SHARED((nacc, cdim // NQ), jnp.float32),
        ],
    )
    return kern(H2, src3, dst3, w3, b)


def kernel(x, edge_index, edge_attr, W, b):
    n, c_in, t_all = x.shape
    e = edge_index.shape[1]

    xT = jnp.transpose(x, (2, 0, 1)).reshape(t_all * n, c_in)
    H = _matmul(xT, W, 2000)

    loop = jnp.arange(n, dtype=edge_index.dtype)
    e_tot = e + n
    grp = NS * CH * KB
    e_pad = -(-e_tot // grp) * grp
    pad = e_pad - e_tot
    src = jnp.concatenate([edge_index[0], loop, jnp.zeros((pad,), jnp.int32)])
    dst = jnp.concatenate([edge_index[1], loop, jnp.zeros((pad,), jnp.int32)])
    w = jnp.concatenate([edge_attr, jnp.ones((n,), jnp.float32),
                         jnp.zeros((pad,), jnp.float32)])
    nchunk = e_pad // (NS * CH)
    src3 = src.reshape(NS, nchunk, CH)
    dst3 = dst.reshape(NS, nchunk, CH)
    w3 = w.reshape(NS, nchunk, CH)

    npad = -(-n // 1024) * 1024
    nacc = -(-(n + 1) // 16) * 16
    c_out = W.shape[1]
    H2 = H.reshape(t_all * n * NQ, c_out // NQ)
    out_sc = _sc_conv(H2, src3, dst3, w3, b, n, npad, nacc, nchunk, t_all)
    out = jnp.transpose(out_sc[:, :, :n, :], (2, 1, 3, 0))
    return out.reshape(n, c_out, t_all)
